# Initial kernel scaffold; baseline (speedup 1.0000x reference)
import jax
import jax.numpy as jnp
from jax.experimental import pallas as pl


def kernel(tree_feature, edge_index, num_tree_nodes, ptr, ln1_g, ln1_b, W_tree, b_tree, ln2_g, ln2_b, W_var, b_var, W_glob, b_glob, glbNode, W_g1, att_src1, att_dst1, bias_g1, W_g2, att_src2, att_dst2, bias_g2):
    raise NotImplementedError("write your pallas kernel here")


# speedup vs baseline: 45.4405x; 45.4405x over previous
import functools
import jax
import jax.numpy as jnp
from jax import lax
from jax.experimental import pallas as pl
from jax.experimental.pallas import tpu as pltpu
from jax.experimental.pallas import tpu_sc as plsc

_N_NODES = 10000
_BATCH = 8
_PER = 1250
_FEAT = 128
_VAR = 25
_HID = 64
_NE = 320000
_N = _N_NODES + _BATCH
_NPAD = 10240
_NPT = _NPAD // 16
_E = _NE + 2 * _PER * _BATCH + _N
_EPAD = 350208
_EPT = _EPAD // 32
_NCH1 = _EPT // 64
_NCH2 = _EPT // 192

_HI = lax.Precision.HIGHEST
_f32 = jnp.float32
_i32 = jnp.int32


def _dot(a, b):
    return jnp.dot(a, b, precision=_HI, preferred_element_type=_f32)


def _pre_body(tf_ref, l1g, l1b, wt, bt, l2g, l2b, wv, bv, wg, bg, out_ref):
    blk = tf_ref[...]
    t = blk[:, :_FEAT]
    v = blk[:, _FEAT:]
    mu = jnp.mean(t, axis=-1, keepdims=True)
    va = jnp.mean((t - mu) ** 2, axis=-1, keepdims=True)
    tn = (t - mu) * lax.rsqrt(va + 1e-5) * l1g[...] + l1b[...]
    mu2 = jnp.mean(v, axis=-1, keepdims=True)
    va2 = jnp.mean((v - mu2) ** 2, axis=-1, keepdims=True)
    vn = (v - mu2) * lax.rsqrt(va2 + 1e-5) * l2g[...] + l2b[...]
    a = _dot(vn, wv[...]) + bv[...]
    b = _dot(tn, wt[...]) + bt[...]
    out_ref[...] = _dot(jnp.concatenate([a, b], axis=1), wg[...]) + bg[...]


def _pre(tfeat, l1g, l1b, wt, bt, l2g, l2b, wv, bv, wg, bg):
    nb = 2000
    full = lambda shp: pl.BlockSpec(shp, lambda i: (0, 0))
    return pl.pallas_call(
        _pre_body,
        grid=(_N_NODES // nb,),
        in_specs=[
            pl.BlockSpec((nb, _FEAT + _VAR), lambda i: (i, 0)),
            full((1, _FEAT)), full((1, _FEAT)),
            full((_FEAT, _HID)), full((1, _HID)),
            full((1, _VAR)), full((1, _VAR)),
            full((_VAR, _HID)), full((1, _HID)),
            full((2 * _HID, _HID)), full((1, _HID)),
        ],
        out_specs=pl.BlockSpec((nb, _HID), lambda i: (i, 0)),
        out_shape=jax.ShapeDtypeStruct((_N_NODES, _HID), _f32),
    )(tfeat, l1g, l1b, wt, bt, l2g, l2b, wv, bv, wg, bg)


def _proj_body(nsteps, x_ref, w_ref, att_ref, xp_ref, st_ref, m_ref, msc):
    i = pl.program_id(0)
    xp = _dot(x_ref[...], w_ref[...])
    xp_ref[...] = xp
    st = _dot(xp, att_ref[...])
    st_ref[...] = st
    cur = jnp.broadcast_to(jnp.max(st, axis=0)[None, :], (8, 4))

    @pl.when(i == 0)
    def _():
        msc[...] = cur

    @pl.when(i > 0)
    def _():
        msc[...] = jnp.maximum(msc[...], cur)

    @pl.when(i == nsteps - 1)
    def _():
        m_ref[...] = msc[...]


def _proj(x, w, att):
    nb = 2560
    nsteps = _NPAD // nb
    din, dout = w.shape
    return pl.pallas_call(
        functools.partial(_proj_body, nsteps),
        grid=(nsteps,),
        in_specs=[
            pl.BlockSpec((nb, din), lambda i: (i, 0)),
            pl.BlockSpec((din, dout), lambda i: (0, 0)),
            pl.BlockSpec((dout, 4), lambda i: (0, 0)),
        ],
        out_specs=[
            pl.BlockSpec((nb, dout), lambda i: (i, 0)),
            pl.BlockSpec((nb, 4), lambda i: (i, 0)),
            pl.BlockSpec((8, 4), lambda i: (0, 0)),
        ],
        out_shape=[
            jax.ShapeDtypeStruct((_NPAD, dout), _f32),
            jax.ShapeDtypeStruct((_NPAD, 4), _f32),
            jax.ShapeDtypeStruct((8, 4), _f32),
        ],
        scratch_shapes=[pltpu.VMEM((8, 4), _f32)],
    )(x, w, att)


def _mid_body(nsteps, a_ref, b_ref, d_ref, b1, w2, att2, xp_ref, st_ref, m_ref, msc):
    i = pl.program_id(0)
    a = a_ref[...]
    b = b_ref[...]
    d = d_ref[...]
    den = d[0] + d[1]
    h = jnp.concatenate(
        [(a[0] + a[1]) / (den[:, 0:1] + 1e-16),
         (b[0] + b[1]) / (den[:, 1:2] + 1e-16)], axis=1)
    h = jnp.maximum(h + b1[...], 0.0)
    xp2 = _dot(h, w2[...])
    xp_ref[...] = xp2
    st = _dot(xp2, att2[...])
    st_ref[...] = st
    cur = jnp.broadcast_to(jnp.max(st, axis=0)[None, :], (8, 4))

    @pl.when(i == 0)
    def _():
        msc[...] = cur

    @pl.when(i > 0)
    def _():
        msc[...] = jnp.maximum(msc[...], cur)

    @pl.when(i == nsteps - 1)
    def _():
        m_ref[...] = msc[...]


def _mid(accpa, accpb, denp, b1, w2, att2):
    nb = 2560
    nsteps = _NPAD // nb
    return pl.pallas_call(
        functools.partial(_mid_body, nsteps),
        grid=(nsteps,),
        in_specs=[
            pl.BlockSpec((2, nb, _HID), lambda i: (0, i, 0)),
            pl.BlockSpec((2, nb, _HID), lambda i: (0, i, 0)),
            pl.BlockSpec((2, nb, 2), lambda i: (0, i, 0)),
            pl.BlockSpec((1, 2 * _HID), lambda i: (0, 0)),
            pl.BlockSpec((2 * _HID, _HID), lambda i: (0, 0)),
            pl.BlockSpec((_HID, 4), lambda i: (0, 0)),
        ],
        out_specs=[
            pl.BlockSpec((nb, _HID), lambda i: (i, 0)),
            pl.BlockSpec((nb, 4), lambda i: (i, 0)),
            pl.BlockSpec((8, 4), lambda i: (0, 0)),
        ],
        out_shape=[
            jax.ShapeDtypeStruct((_NPAD, _HID), _f32),
            jax.ShapeDtypeStruct((_NPAD, 4), _f32),
            jax.ShapeDtypeStruct((8, 4), _f32),
        ],
        scratch_shapes=[pltpu.VMEM((8, 4), _f32)],
    )(accpa, accpb, denp, b1, w2, att2)


def _glb_body(ex_ref, xg_ref, xs_ref, es_ref, dg_ref, b2, out_ref):
    ex = ex_ref[...]
    xg = xg_ref[...]
    num = jnp.concatenate(
        [_dot(ex[b:b + 1, :], xg[b]) for b in range(_BATCH)], axis=0)
    num = num + es_ref[...] * xs_ref[...]
    out_ref[...] = num / (dg_ref[...] + 1e-16) + b2[...]


def _glb(ex2g, xg, xself, exself, dgb, b2):
    return pl.pallas_call(
        _glb_body,
        out_shape=jax.ShapeDtypeStruct((_BATCH, 128), _f32),
    )(ex2g, xg, xself, exself, dgb, b2)


_NST = 10016
_P1C = 912


def _edge1_body(srcs, dstsf, stats1, m1f, xp1a, xp1b,
                accpa, accpb, denp1, denall,
                stats_v, m_v, sidx1_v, didx1_v, sidx_v, didx_v, den_v,
                ex0_v, ex1_v, rows_v, red_v, tmp_v,
                acc_sh, sem, sem2):
    c = lax.axis_index("c")
    s = lax.axis_index("s")
    wid = c * 16 + s
    ebase = wid * _EPT
    pltpu.sync_copy(stats1, stats_v)
    pltpu.sync_copy(m1f, m_v)
    m0 = m_v[pl.ds(0, 16)]
    m1 = m_v[pl.ds(16, 16)]
    z16 = jnp.zeros((16,), _f32)
    for k in range(64):
        for f in range(4):
            rows_v[k, pl.ds(f * 16, 16)] = z16

    def zden(i, carry):
        den_v[pl.ds(i * 16, 16)] = z16
        return carry
    lax.fori_loop(0, 2 * _NPAD // 16, zden, 0)

    def zacc():
        for k in range(_NPT // 64):
            pltpu.sync_copy(rows_v, acc_sh.at[pl.ds(s * _NPT + k * 64, 64)])
    zacc()
    plsc.subcore_barrier()

    def exchunk(j, carry):
        base = j * _P1C
        pltpu.sync_copy(srcs.at[pl.ds(ebase + base, _P1C)], sidx1_v)
        pltpu.sync_copy(dstsf.at[pl.ds(ebase + base, _P1C)], didx1_v)

        def grp(g, carry2):
            off = g * 16
            sv = sidx1_v[pl.ds(off, 16)]
            dv = didx1_v[pl.ds(off, 16)]
            sv4 = sv * 4
            dv4 = dv * 4
            a0 = plsc.load_gather(stats_v, [sv4])
            a1 = plsc.load_gather(stats_v, [sv4 + 1])
            d0 = plsc.load_gather(stats_v, [dv4 + 2])
            d1 = plsc.load_gather(stats_v, [dv4 + 3])
            l0 = a0 + d0
            l0 = jnp.where(l0 > 0, l0, l0 * 0.2)
            e0 = jnp.exp(l0 - jnp.maximum(m0 + d0, 0.0))
            l1 = a1 + d1
            l1 = jnp.where(l1 > 0, l1, l1 * 0.2)
            e1 = jnp.exp(l1 - jnp.maximum(m1 + d1, 0.0))
            eoff = base + off
            ex0_v[pl.ds(eoff, 16)] = e0
            ex1_v[pl.ds(eoff, 16)] = e1
            di = dv * 2
            plsc.addupdate_scatter(den_v, [di], e0)
            plsc.addupdate_scatter(den_v, [di + 1], e1)
            return carry2

        lax.fori_loop(0, _P1C // 16, grp, 0)
        return carry

    lax.fori_loop(0, _EPT // _P1C, exchunk, 0)

    def half_pass(table, ex_v, accp):
        def chunk(j, carry):
            base = j * 64
            pltpu.sync_copy(srcs.at[pl.ds(ebase + base, 64)], sidx_v)
            pltpu.sync_copy(dstsf.at[pl.ds(ebase + base, 64)], didx_v)
            pltpu.async_copy(table.at[sidx_v], rows_v, sem).wait()
            for g in range(4):
                ev = ex_v[pl.ds(base + g * 16, 16)]
                for kk in range(16):
                    k = g * 16 + kk
                    eb = lax.broadcast(ev[kk], (16,))
                    for f in range(4):
                        sl = pl.ds(f * 16, 16)
                        rows_v[k, sl] = rows_v[k, sl] * eb
            pltpu.async_copy(rows_v, acc_sh.at[didx_v], sem2,
                             add=True).wait()
            return carry

        lax.fori_loop(0, _NCH1, chunk, 0)
        plsc.subcore_barrier()
        pltpu.sync_copy(acc_sh.at[pl.ds(s * _NPT, _NPT)],
                        accp.at[c, pl.ds(s * _NPT, _NPT)])

    half_pass(xp1a, ex0_v, accpa)
    for k in range(64):
        for f in range(4):
            rows_v[k, pl.ds(f * 16, 16)] = z16
    zacc()
    plsc.subcore_barrier()
    half_pass(xp1b, ex1_v, accpb)

    pltpu.sync_copy(den_v, denall.at[c, s])
    plsc.subcore_barrier()
    for half in range(2):
        hb = s * 1280 + half * 640
        for i in range(40):
            red_v[pl.ds(i * 16, 16)] = z16

        def red_t(t, carry):
            pltpu.sync_copy(denall.at[c, t, pl.ds(hb, 640)], tmp_v)
            for i in range(40):
                sl = pl.ds(i * 16, 16)
                red_v[sl] = red_v[sl] + tmp_v[sl]
            return carry

        lax.fori_loop(0, 16, red_t, 0)
        pltpu.sync_copy(red_v, denp1.at[c, pl.ds(hb, 640)])


def _edge1(srcs, dsts, stats1, m1f, xp1a, xp1b):
    mesh = plsc.VectorSubcoreMesh(core_axis_name="c", subcore_axis_name="s")
    f = pl.kernel(
        _edge1_body,
        out_type=[
            jax.ShapeDtypeStruct((2, _NPAD, _HID), _f32),
            jax.ShapeDtypeStruct((2, _NPAD, _HID), _f32),
            jax.ShapeDtypeStruct((2, 2 * _NPAD), _f32),
            jax.ShapeDtypeStruct((2, 16, 2 * _NPAD), _f32),
        ],
        mesh=mesh,
        compiler_params=pltpu.CompilerParams(needs_layout_passes=False, use_tc_tiling_on_sc=False),
        scratch_types=[
            pltpu.VMEM((_NST * 4,), _f32),
            pltpu.VMEM((32,), _f32),
            pltpu.VMEM((_P1C,), _i32),
            pltpu.VMEM((_P1C,), _i32),
            pltpu.VMEM((64,), _i32),
            pltpu.VMEM((64,), _i32),
            pltpu.VMEM((2 * _NPAD,), _f32),
            pltpu.VMEM((_EPT,), _f32),
            pltpu.VMEM((_EPT,), _f32),
            pltpu.VMEM((64, _HID), _f32),
            pltpu.VMEM((640,), _f32),
            pltpu.VMEM((640,), _f32),
            pltpu.VMEM_SHARED((_NPAD, _HID), _f32),
            pltpu.SemaphoreType.DMA,
            pltpu.SemaphoreType.DMA,
        ],
    )
    return f(srcs, dsts, stats1, m1f, xp1a, xp1b)


def _edge2_body(srcs, dstsf, stats2, m2f,
                ex2, denp2, denall,
                stats_v, m_v, src_v, dstf_v, den_v, exc_v, red_v, tmp_v,
                sem):
    c = lax.axis_index("c")
    s = lax.axis_index("s")
    wid = c * 16 + s
    pltpu.sync_copy(stats2, stats_v)
    pltpu.sync_copy(m2f, m_v)
    pltpu.sync_copy(srcs.at[pl.ds(wid * _EPT, _EPT)], src_v)
    pltpu.sync_copy(dstsf.at[pl.ds(wid * _EPT, _EPT)], dstf_v)
    m2 = m_v[pl.ds(0, 16)]
    z16 = jnp.zeros((16,), _f32)

    def zden(i, carry):
        den_v[pl.ds(i * 16, 16)] = z16
        return carry
    lax.fori_loop(0, _NPAD // 16, zden, 0)

    def chunk(j, carry):
        base = j * 192
        for g in range(12):
            off = base + g * 16
            sv = src_v[pl.ds(off, 16)]
            dv = dstf_v[pl.ds(off, 16)]
            a = plsc.load_gather(stats_v, [sv * 4])
            d = plsc.load_gather(stats_v, [dv * 4 + 1])
            l = a + d
            l = jnp.where(l > 0, l, l * 0.2)
            e = jnp.exp(l - jnp.maximum(m2 + d, 0.0))
            exc_v[pl.ds(g * 16, 16)] = e
            plsc.addupdate_scatter(den_v, [dv], e)
        pltpu.sync_copy(exc_v, ex2.at[pl.ds(wid * _EPT + base, 192)])
        return carry

    lax.fori_loop(0, _NCH2, chunk, 0)
    pltpu.sync_copy(den_v, denall.at[c, s])
    plsc.subcore_barrier()
    for i in range(_NPT // 16):
        red_v[pl.ds(i * 16, 16)] = z16

    def red_t(t, carry):
        pltpu.sync_copy(denall.at[c, t, pl.ds(s * _NPT, _NPT)], tmp_v)
        for i in range(_NPT // 16):
            sl = pl.ds(i * 16, 16)
            red_v[sl] = red_v[sl] + tmp_v[sl]
        return carry

    lax.fori_loop(0, 16, red_t, 0)
    pltpu.sync_copy(red_v, denp2.at[c, pl.ds(s * _NPT, _NPT)])


def _edge2(srcs, dsts, stats2, m2f):
    mesh = plsc.VectorSubcoreMesh(core_axis_name="c", subcore_axis_name="s")
    f = pl.kernel(
        _edge2_body,
        out_type=[
            jax.ShapeDtypeStruct((_EPAD,), _f32),
            jax.ShapeDtypeStruct((2, _NPAD), _f32),
            jax.ShapeDtypeStruct((2, 16, _NPAD), _f32),
        ],
        mesh=mesh,
        compiler_params=pltpu.CompilerParams(needs_layout_passes=False, use_tc_tiling_on_sc=False),
        scratch_types=[
            pltpu.VMEM((_NPAD * 4,), _f32),
            pltpu.VMEM((16,), _f32),
            pltpu.VMEM((_EPT,), _i32),
            pltpu.VMEM((_EPT,), _i32),
            pltpu.VMEM((_NPAD,), _f32),
            pltpu.VMEM((192,), _f32),
            pltpu.VMEM((_NPT,), _f32),
            pltpu.VMEM((_NPT,), _f32),
            pltpu.SemaphoreType.DMA,
        ],
    )
    return f(srcs, dsts, stats2, m2f)


def _alpha_body(ex2, dstsf, denp2,
                alpha, dentot,
                den_v, tmp_full, dstf_v, exc_v, alc_v, sem):
    c = lax.axis_index("c")
    s = lax.axis_index("s")
    wid = c * 16 + s
    pltpu.sync_copy(denp2.at[0], den_v)
    pltpu.sync_copy(denp2.at[1], tmp_full)

    def addr(i, carry):
        sl = pl.ds(i * 16, 16)
        den_v[sl] = den_v[sl] + tmp_full[sl]
        return carry

    lax.fori_loop(0, _NPAD // 16, addr, 0)

    @pl.when(wid == 0)
    def _():
        pltpu.sync_copy(den_v, dentot)

    pltpu.sync_copy(dstsf.at[pl.ds(wid * _EPT, _EPT)], dstf_v)

    def chunk(j, carry):
        base = j * 192
        pltpu.sync_copy(ex2.at[pl.ds(wid * _EPT + base, 192)], exc_v)
        for g in range(12):
            sl = pl.ds(g * 16, 16)
            dv = dstf_v[pl.ds(base + g * 16, 16)]
            dd = plsc.load_gather(den_v, [dv])
            alc_v[sl] = exc_v[sl] / (dd + 1e-16)
        pltpu.sync_copy(alc_v, alpha.at[pl.ds(wid * _EPT + base, 192)])
        return carry

    lax.fori_loop(0, _NCH2, chunk, 0)


def _alpha_k(ex2, dsts, denp2):
    mesh = plsc.VectorSubcoreMesh(core_axis_name="c", subcore_axis_name="s")
    f = pl.kernel(
        _alpha_body,
        out_type=[
            jax.ShapeDtypeStruct((_EPAD,), _f32),
            jax.ShapeDtypeStruct((_NPAD,), _f32),
        ],
        mesh=mesh,
        compiler_params=pltpu.CompilerParams(needs_layout_passes=False, use_tc_tiling_on_sc=False),
        scratch_types=[
            pltpu.VMEM((_NPAD,), _f32),
            pltpu.VMEM((_NPAD,), _f32),
            pltpu.VMEM((_EPT,), _i32),
            pltpu.VMEM((192,), _f32),
            pltpu.VMEM((192,), _f32),
            pltpu.SemaphoreType.DMA,
        ],
    )
    return f(ex2, dsts, denp2)


def kernel(tree_feature, edge_index, num_tree_nodes, ptr,
           ln1_g, ln1_b, W_tree, b_tree, ln2_g, ln2_b, W_var, b_var,
           W_glob, b_glob, glbNode, W_g1, att_src1, att_dst1, bias_g1,
           W_g2, att_src2, att_dst2, bias_g2):
    loop = jnp.arange(_N, dtype=_i32)
    parts = []
    for b in range(_BATCH):
        nodes = ptr[b] + jnp.arange(_PER, dtype=_i32)
        g = jnp.full((_PER,), _N_NODES + b, dtype=_i32)
        parts.append(jnp.stack([g, nodes]))
        parts.append(jnp.stack([nodes, g]))
    ei = jnp.concatenate(
        [edge_index.astype(_i32)] + parts + [jnp.stack([loop, loop])], axis=1)
    padv = jnp.full((_EPAD - _E,), _NST - 1, dtype=_i32)
    srcs = jnp.concatenate([ei[0], padv])
    dsts = jnp.concatenate([ei[1], padv])

    r1 = lambda a: a.reshape(1, -1)
    x0 = _pre(tree_feature, r1(ln1_g), r1(ln1_b), W_tree, r1(b_tree),
              r1(ln2_g), r1(ln2_b), W_var, r1(b_var), W_glob, r1(b_glob))
    x = jnp.concatenate(
        [x0, jnp.broadcast_to(glbNode[None], (_BATCH, _HID)),
         jnp.zeros((_NPAD - _N, _HID), _f32)], axis=0)

    z64 = jnp.zeros((_HID,), _f32)
    att1 = jnp.stack([
        jnp.concatenate([att_src1[0], z64]),
        jnp.concatenate([z64, att_src1[1]]),
        jnp.concatenate([att_dst1[0], z64]),
        jnp.concatenate([z64, att_dst1[1]]),
    ], axis=1)
    xp1, stats1, m1 = _proj(x, W_g1, att1)
    m1f = jnp.concatenate([jnp.broadcast_to(m1[0, 0], (16,)),
                           jnp.broadcast_to(m1[0, 1], (16,))])

    xp1a = xp1[:, :_HID]
    xp1b = xp1[:, _HID:]
    accpa, accpb, denp1, _da = _edge1(srcs, dsts,
                                      stats1[:_NST].reshape(-1), m1f,
                                      xp1a, xp1b)

    att2 = jnp.stack([att_src2[0], att_dst2[0], z64, z64], axis=1)
    xp2, stats2, m2 = _mid(accpa, accpb, denp1.reshape(2, _NPAD, 2),
                           r1(bias_g1), W_g2, att2)
    m2f = jnp.broadcast_to(m2[0, 0], (16,))

    ex2, denp2, _db = _edge2(srcs, dsts, stats2.reshape(-1), m2f)
    alpha_full, dentot = _alpha_k(ex2, dsts, denp2)
    alpha = alpha_full[:_E][:, None]

    ngl = 2 * _PER * _BATCH
    ex2g = ex2[_NE:_NE + ngl].reshape(_BATCH, 2, _PER)[:, 1, :]
    ex2g = jnp.pad(ex2g, ((0, 0), (0, 30)))
    ex2self = ex2[_NE + ngl + _N_NODES:_NE + ngl + _N]
    xg = jnp.pad(xp2[:_N_NODES].reshape(_BATCH, _PER, _HID),
                 ((0, 0), (0, 30), (0, 64)))
    xself = jnp.pad(xp2[_N_NODES:_N], ((0, 0), (0, 64)))
    exselfb = jnp.broadcast_to(ex2self[:, None], (_BATCH, 128))
    dgb = jnp.broadcast_to(dentot[_N_NODES:_N][:, None], (_BATCH, 128))
    b2 = jnp.pad(bias_g2, (0, 64)).reshape(1, 128)
    tg = _glb(ex2g, xg, xself, exselfb, dgb, b2)
    tree_glb = tg[:, :_HID]
    return (tree_glb, (ei, alpha))


# speedup vs baseline: 95.2328x; 2.0958x over previous
import functools
import jax
import jax.numpy as jnp
from jax import lax
from jax.experimental import pallas as pl
from jax.experimental.pallas import tpu as pltpu
from jax.experimental.pallas import tpu_sc as plsc

_N_NODES = 10000
_BATCH = 8
_PER = 1250
_FEAT = 128
_VAR = 25
_HID = 64
_NE = 320000
_N = _N_NODES + _BATCH
_NPAD = 10240
_NPT = _NPAD // 16
_E = _NE + 2 * _PER * _BATCH + _N
_EPAD = 350208
_EPT = _EPAD // 32
_NCH1 = _EPT // 64
_P2C = 1824
_NCH2 = _EPT // _P2C

_HI = lax.Precision.HIGHEST
_f32 = jnp.float32
_i32 = jnp.int32


def _splat16(v, idx):
    dn = lax.GatherDimensionNumbers(offset_dims=(), collapsed_slice_dims=(0,),
                                    start_index_map=(0,))
    return lax.gather(v, idx[:, None], dimension_numbers=dn, slice_sizes=(1,),
                      mode=lax.GatherScatterMode.PROMISE_IN_BOUNDS)


def _dot(a, b):
    return jnp.dot(a, b, precision=_HI, preferred_element_type=_f32)


def _pre_body(tf_ref, l1g, l1b, wt, bt, l2g, l2b, wv, bv, wg, bg, out_ref):
    blk = tf_ref[...]
    t = blk[:, :_FEAT]
    v = blk[:, _FEAT:]
    mu = jnp.mean(t, axis=-1, keepdims=True)
    va = jnp.mean((t - mu) ** 2, axis=-1, keepdims=True)
    tn = (t - mu) * lax.rsqrt(va + 1e-5) * l1g[...] + l1b[...]
    mu2 = jnp.mean(v, axis=-1, keepdims=True)
    va2 = jnp.mean((v - mu2) ** 2, axis=-1, keepdims=True)
    vn = (v - mu2) * lax.rsqrt(va2 + 1e-5) * l2g[...] + l2b[...]
    a = _dot(vn, wv[...]) + bv[...]
    b = _dot(tn, wt[...]) + bt[...]
    out_ref[...] = _dot(jnp.concatenate([a, b], axis=1), wg[...]) + bg[...]


def _pre(tfeat, l1g, l1b, wt, bt, l2g, l2b, wv, bv, wg, bg):
    nb = 2000
    full = lambda shp: pl.BlockSpec(shp, lambda i: (0, 0))
    return pl.pallas_call(
        _pre_body,
        grid=(_N_NODES // nb,),
        in_specs=[
            pl.BlockSpec((nb, _FEAT + _VAR), lambda i: (i, 0)),
            full((1, _FEAT)), full((1, _FEAT)),
            full((_FEAT, _HID)), full((1, _HID)),
            full((1, _VAR)), full((1, _VAR)),
            full((_VAR, _HID)), full((1, _HID)),
            full((2 * _HID, _HID)), full((1, _HID)),
        ],
        out_specs=pl.BlockSpec((nb, _HID), lambda i: (i, 0)),
        out_shape=jax.ShapeDtypeStruct((_N_NODES, _HID), _f32),
    )(tfeat, l1g, l1b, wt, bt, l2g, l2b, wv, bv, wg, bg)


def _proj_body(nsteps, x_ref, w_ref, att_ref, xp_ref, st_ref, m_ref, msc):
    i = pl.program_id(0)
    xp = _dot(x_ref[...], w_ref[...])
    xp_ref[...] = xp
    st = _dot(xp, att_ref[...])
    st_ref[...] = st
    cur = jnp.broadcast_to(jnp.max(st, axis=0)[None, :], (8, 4))

    @pl.when(i == 0)
    def _():
        msc[...] = cur

    @pl.when(i > 0)
    def _():
        msc[...] = jnp.maximum(msc[...], cur)

    @pl.when(i == nsteps - 1)
    def _():
        m_ref[...] = msc[...]


def _proj(x, w, att):
    nb = 2560
    nsteps = _NPAD // nb
    din, dout = w.shape
    return pl.pallas_call(
        functools.partial(_proj_body, nsteps),
        grid=(nsteps,),
        in_specs=[
            pl.BlockSpec((nb, din), lambda i: (i, 0)),
            pl.BlockSpec((din, dout), lambda i: (0, 0)),
            pl.BlockSpec((dout, 4), lambda i: (0, 0)),
        ],
        out_specs=[
            pl.BlockSpec((nb, dout), lambda i: (i, 0)),
            pl.BlockSpec((nb, 4), lambda i: (i, 0)),
            pl.BlockSpec((8, 4), lambda i: (0, 0)),
        ],
        out_shape=[
            jax.ShapeDtypeStruct((_NPAD, dout), _f32),
            jax.ShapeDtypeStruct((_NPAD, 4), _f32),
            jax.ShapeDtypeStruct((8, 4), _f32),
        ],
        scratch_shapes=[pltpu.VMEM((8, 4), _f32)],
    )(x, w, att)


def _mid_body(nsteps, a_ref, d_ref, b1, w2, att2, xp_ref, st_ref, m_ref, msc):
    i = pl.program_id(0)
    a = a_ref[...]
    acc = a[0] + a[1]
    d = d_ref[...]
    den = d[0] + d[1]
    h = jnp.concatenate(
        [acc[:, :_HID] / (den[:, 0:1] + 1e-16),
         acc[:, _HID:] / (den[:, 1:2] + 1e-16)], axis=1)
    h = jnp.maximum(h + b1[...], 0.0)
    xp2 = _dot(h, w2[...])
    xp_ref[...] = xp2
    st = _dot(xp2, att2[...])
    st_ref[...] = st
    cur = jnp.broadcast_to(jnp.max(st, axis=0)[None, :], (8, 4))

    @pl.when(i == 0)
    def _():
        msc[...] = cur

    @pl.when(i > 0)
    def _():
        msc[...] = jnp.maximum(msc[...], cur)

    @pl.when(i == nsteps - 1)
    def _():
        m_ref[...] = msc[...]


def _mid(accp, denp, b1, w2, att2):
    nb = 2560
    nsteps = _NPAD // nb
    return pl.pallas_call(
        functools.partial(_mid_body, nsteps),
        grid=(nsteps,),
        in_specs=[
            pl.BlockSpec((2, nb, 2 * _HID), lambda i: (0, i, 0)),
            pl.BlockSpec((2, nb, 2), lambda i: (0, i, 0)),
            pl.BlockSpec((1, 2 * _HID), lambda i: (0, 0)),
            pl.BlockSpec((2 * _HID, _HID), lambda i: (0, 0)),
            pl.BlockSpec((_HID, 4), lambda i: (0, 0)),
        ],
        out_specs=[
            pl.BlockSpec((nb, _HID), lambda i: (i, 0)),
            pl.BlockSpec((nb, 4), lambda i: (i, 0)),
            pl.BlockSpec((8, 4), lambda i: (0, 0)),
        ],
        out_shape=[
            jax.ShapeDtypeStruct((_NPAD, _HID), _f32),
            jax.ShapeDtypeStruct((_NPAD, 4), _f32),
            jax.ShapeDtypeStruct((8, 4), _f32),
        ],
        scratch_shapes=[pltpu.VMEM((8, 4), _f32)],
    )(accp, denp, b1, w2, att2)


def _glb_body(ex_ref, xg_ref, xs_ref, es_ref, dg_ref, b2, out_ref):
    ex = ex_ref[...]
    xg = xg_ref[...]
    num = jnp.concatenate(
        [_dot(ex[b:b + 1, :], xg[b]) for b in range(_BATCH)], axis=0)
    num = num + es_ref[...] * xs_ref[...]
    out_ref[...] = num / (dg_ref[...] + 1e-16) + b2[...]


def _glb(ex2g, xg, xself, exself, dgb, b2):
    return pl.pallas_call(
        _glb_body,
        out_shape=jax.ShapeDtypeStruct((_BATCH, 128), _f32),
    )(ex2g, xg, xself, exself, dgb, b2)


_NST = 10016
_P1C = 5472
_AGC = 96
_NAG = _EPT // _AGC
_NC0 = 146
_NC1 = 2 * _NAG - _NC0


def _ex1_body(srcs, dstsf, stats1, m1f,
              exw, denp1, denall,
              stats_v, m_v, sidx1_v, didx1_v, exc0_v, exc1_v, den_v,
              red_v, tmp_v, sem):
    c = lax.axis_index("c")
    s = lax.axis_index("s")
    wid = c * 16 + s
    ebase = wid * _EPT
    pltpu.sync_copy(stats1, stats_v)
    pltpu.sync_copy(m1f, m_v)
    m0 = m_v[pl.ds(0, 16)]
    m1 = m_v[pl.ds(16, 16)]
    z16 = jnp.zeros((16,), _f32)

    def zden(i, carry):
        den_v[pl.ds(i * 16, 16)] = z16
        return carry
    lax.fori_loop(0, 2 * _NPAD // 16, zden, 0)

    def exchunk(j, carry):
        base = j * _P1C
        pltpu.sync_copy(srcs.at[pl.ds(ebase + base, _P1C)], sidx1_v)
        pltpu.sync_copy(dstsf.at[pl.ds(ebase + base, _P1C)], didx1_v)

        def grp(g, carry2):
            off = g * 16
            sv = sidx1_v[pl.ds(off, 16)]
            dv = didx1_v[pl.ds(off, 16)]
            sv4 = sv * 4
            dv4 = dv * 4
            a0 = plsc.load_gather(stats_v, [sv4])
            a1 = plsc.load_gather(stats_v, [sv4 + 1])
            d0 = plsc.load_gather(stats_v, [dv4 + 2])
            d1 = plsc.load_gather(stats_v, [dv4 + 3])
            l0 = a0 + d0
            l0 = jnp.where(l0 > 0, l0, l0 * 0.2)
            e0 = jnp.exp(l0 - jnp.maximum(m0 + d0, 0.0))
            l1 = a1 + d1
            l1 = jnp.where(l1 > 0, l1, l1 * 0.2)
            e1 = jnp.exp(l1 - jnp.maximum(m1 + d1, 0.0))
            exc0_v[pl.ds(off, 16)] = e0
            exc1_v[pl.ds(off, 16)] = e1
            di = dv * 2
            plsc.addupdate_scatter(den_v, [di], e0)
            plsc.addupdate_scatter(den_v, [di + 1], e1)
            return carry2

        lax.fori_loop(0, _P1C // 16, grp, 0)
        pltpu.sync_copy(exc0_v, exw.at[0, pl.ds(ebase + base, _P1C)])
        pltpu.sync_copy(exc1_v, exw.at[1, pl.ds(ebase + base, _P1C)])
        return carry

    lax.fori_loop(0, _EPT // _P1C, exchunk, 0)

    pltpu.sync_copy(den_v, denall.at[c, s])
    plsc.subcore_barrier()
    for half in range(2):
        hb = s * 1280 + half * 640
        for i in range(40):
            red_v[pl.ds(i * 16, 16)] = z16

        def red_t(t, carry):
            pltpu.sync_copy(denall.at[c, t, pl.ds(hb, 640)], tmp_v)
            for i in range(40):
                sl = pl.ds(i * 16, 16)
                red_v[sl] = red_v[sl] + tmp_v[sl]
            return carry

        lax.fori_loop(0, 16, red_t, 0)
        pltpu.sync_copy(red_v, denp1.at[c, pl.ds(hb, 640)])


def _ex1(srcs, dsts, stats1, m1f):
    mesh = plsc.VectorSubcoreMesh(core_axis_name="c", subcore_axis_name="s")
    f = pl.kernel(
        _ex1_body,
        out_type=[
            jax.ShapeDtypeStruct((2, _EPAD), _f32),
            jax.ShapeDtypeStruct((2, 2 * _NPAD), _f32),
            jax.ShapeDtypeStruct((2, 16, 2 * _NPAD), _f32),
        ],
        mesh=mesh,
        compiler_params=pltpu.CompilerParams(
            needs_layout_passes=False, use_tc_tiling_on_sc=False),
        scratch_types=[
            pltpu.VMEM((_NST * 4,), _f32),
            pltpu.VMEM((32,), _f32),
            pltpu.VMEM((_P1C,), _i32),
            pltpu.VMEM((_P1C,), _i32),
            pltpu.VMEM((_P1C,), _f32),
            pltpu.VMEM((_P1C,), _f32),
            pltpu.VMEM((2 * _NPAD,), _f32),
            pltpu.VMEM((640,), _f32),
            pltpu.VMEM((640,), _f32),
            pltpu.SemaphoreType.DMA,
        ],
    )
    return f(srcs, dsts, stats1, m1f)


def _agg_body(srcs, dstsf, exw, xp1,
              accp,
              sA, dA, e0A, e1A, sB, dB, e0B, e1B, dsA, dsB,
              rowsA, rowsB,
              acc_sh,
              sem_iA, sem_iB, sem_gA, sem_gB, sem_sA, sem_sB):
    c = lax.axis_index("c")
    s = lax.axis_index("s")
    nch = jnp.where(c == 0, _NC0, _NC1)
    ebase = jnp.where(c == 0, s * (_NC0 * _AGC),
                      16 * (_NC0 * _AGC) + s * (_NC1 * _AGC))
    z16 = jnp.zeros((16,), _f32)

    for k in range(_AGC):
        for f in range(8):
            rowsA[k, pl.ds(f * 16, 16)] = z16
    nfull = _NPT // _AGC
    for k in range(nfull):
        pltpu.sync_copy(rowsA, acc_sh.at[pl.ds(s * _NPT + k * _AGC, _AGC)])
    rem = _NPT - nfull * _AGC
    if rem:
        pltpu.sync_copy(rowsA.at[pl.ds(0, rem)],
                        acc_sh.at[pl.ds(s * _NPT + nfull * _AGC, rem)])
    plsc.subcore_barrier()

    bufs = ((sA, dA, e0A, e1A, dsA, rowsA, sem_iA, sem_gA, sem_sA),
            (sB, dB, e0B, e1B, dsB, rowsB, sem_iB, sem_gB, sem_sB))
    kkc = [jnp.full((16,), t, _i32) for t in range(16)]

    def issue_idx(P, j):
        (si, di, e0, e1, _, _, sem_i, _, _) = bufs[P]
        off = ebase + j * _AGC
        pltpu.async_copy(srcs.at[pl.ds(off, _AGC)], si, sem_i)
        pltpu.async_copy(dstsf.at[pl.ds(off, _AGC)], di, sem_i)
        pltpu.async_copy(exw.at[0, pl.ds(off, _AGC)], e0, sem_i)
        pltpu.async_copy(exw.at[1, pl.ds(off, _AGC)], e1, sem_i)

    def wait_idx(P, j):
        (si, di, e0, e1, _, _, sem_i, _, _) = bufs[P]
        off = ebase + j * _AGC
        pltpu.make_async_copy(srcs.at[pl.ds(off, _AGC)], si, sem_i).wait()
        pltpu.make_async_copy(dstsf.at[pl.ds(off, _AGC)], di, sem_i).wait()
        pltpu.make_async_copy(exw.at[0, pl.ds(off, _AGC)], e0, sem_i).wait()
        pltpu.make_async_copy(exw.at[1, pl.ds(off, _AGC)], e1, sem_i).wait()

    def issue_gather(P):
        (si, _, _, _, _, rows, _, sem_g, _) = bufs[P]
        pltpu.async_copy(xp1.at[si], rows, sem_g)

    def wait_gather(P):
        (si, _, _, _, _, rows, _, sem_g, _) = bufs[P]
        pltpu.make_async_copy(xp1.at[si], rows, sem_g).wait()

    def issue_scatter(P):
        (_, _, _, _, dsc, rows, _, _, sem_s) = bufs[P]
        pltpu.async_copy(rows, acc_sh.at[dsc], sem_s, add=True)

    def wait_scatter(P):
        (_, _, _, _, dsc, rows, _, _, sem_s) = bufs[P]
        pltpu.make_async_copy(rows, acc_sh.at[dsc], sem_s).wait()

    def sub(P, j):
        (si, di, e0, e1, dsc, rows, _, _, _) = bufs[P]
        Q = 1 - P

        @pl.when(j >= 1)
        def _():
            wait_scatter(Q)

        @pl.when(j <= nch - 2)
        def _():
            wait_idx(Q, j + 1)
            issue_gather(Q)

        wait_gather(P)
        for g in range(_AGC // 16):
            ev0 = e0[pl.ds(g * 16, 16)]
            ev1 = e1[pl.ds(g * 16, 16)]
            for kk in range(16):
                k = g * 16 + kk
                eb0 = _splat16(ev0, kkc[kk])
                eb1 = _splat16(ev1, kkc[kk])
                for f in range(4):
                    sl = pl.ds(f * 16, 16)
                    rows[k, sl] = rows[k, sl] * eb0
                for f in range(4, 8):
                    sl = pl.ds(f * 16, 16)
                    rows[k, sl] = rows[k, sl] * eb1
        for g in range(_AGC // 16):
            sl = pl.ds(g * 16, 16)
            dsc[sl] = di[sl]
        issue_scatter(P)

        @pl.when(j <= nch - 3)
        def _():
            issue_idx(P, j + 2)

    issue_idx(0, 0)
    issue_idx(1, 1)
    wait_idx(0, 0)
    issue_gather(0)

    def pair(i, carry):
        sub(0, 2 * i)
        sub(1, 2 * i + 1)
        return carry

    lax.fori_loop(0, nch // 2, pair, 0)
    wait_scatter(1)
    plsc.subcore_barrier()
    pltpu.sync_copy(acc_sh.at[pl.ds(s * _NPT, _NPT)],
                    accp.at[c, pl.ds(s * _NPT, _NPT)])


def _agg(srcs, dsts, exw, xp1):
    mesh = plsc.VectorSubcoreMesh(core_axis_name="c", subcore_axis_name="s")
    f = pl.kernel(
        _agg_body,
        out_type=jax.ShapeDtypeStruct((2, _NPAD, 2 * _HID), _f32),
        mesh=mesh,
        compiler_params=pltpu.CompilerParams(
            needs_layout_passes=False, use_tc_tiling_on_sc=False),
        scratch_types=[
            pltpu.VMEM((_AGC,), _i32), pltpu.VMEM((_AGC,), _i32),
            pltpu.VMEM((_AGC,), _f32), pltpu.VMEM((_AGC,), _f32),
            pltpu.VMEM((_AGC,), _i32), pltpu.VMEM((_AGC,), _i32),
            pltpu.VMEM((_AGC,), _f32), pltpu.VMEM((_AGC,), _f32),
            pltpu.VMEM((_AGC,), _i32), pltpu.VMEM((_AGC,), _i32),
            pltpu.VMEM((_AGC, 2 * _HID), _f32),
            pltpu.VMEM((_AGC, 2 * _HID), _f32),
            pltpu.VMEM_SHARED((_NPAD, 2 * _HID), _f32),
            pltpu.SemaphoreType.DMA, pltpu.SemaphoreType.DMA,
            pltpu.SemaphoreType.DMA, pltpu.SemaphoreType.DMA,
            pltpu.SemaphoreType.DMA, pltpu.SemaphoreType.DMA,
        ],
    )
    return f(srcs, dsts, exw, xp1)


def _edge2_body(srcs, dstsf, stats2, m2f,
                ex2, denp2, denall,
                stats_v, m_v, src_v, dstf_v, den_v, exc_v, red_v, tmp_v,
                sem):
    c = lax.axis_index("c")
    s = lax.axis_index("s")
    wid = c * 16 + s
    pltpu.sync_copy(stats2, stats_v)
    pltpu.sync_copy(m2f, m_v)
    pltpu.sync_copy(srcs.at[pl.ds(wid * _EPT, _EPT)], src_v)
    pltpu.sync_copy(dstsf.at[pl.ds(wid * _EPT, _EPT)], dstf_v)
    m2 = m_v[pl.ds(0, 16)]
    z16 = jnp.zeros((16,), _f32)

    def zden(i, carry):
        den_v[pl.ds(i * 16, 16)] = z16
        return carry
    lax.fori_loop(0, _NPAD // 16, zden, 0)

    def chunk(j, carry):
        base = j * _P2C

        def grp(g, carry2):
            off = base + g * 16
            sv = src_v[pl.ds(off, 16)]
            dv = dstf_v[pl.ds(off, 16)]
            a = plsc.load_gather(stats_v, [sv * 4])
            d = plsc.load_gather(stats_v, [dv * 4 + 1])
            l = a + d
            l = jnp.where(l > 0, l, l * 0.2)
            e = jnp.exp(l - jnp.maximum(m2 + d, 0.0))
            exc_v[pl.ds(g * 16, 16)] = e
            plsc.addupdate_scatter(den_v, [dv], e)
            return carry2

        lax.fori_loop(0, _P2C // 16, grp, 0)
        pltpu.sync_copy(exc_v, ex2.at[pl.ds(wid * _EPT + base, _P2C)])
        return carry

    lax.fori_loop(0, _NCH2, chunk, 0)
    pltpu.sync_copy(den_v, denall.at[c, s])
    plsc.subcore_barrier()
    for i in range(_NPT // 16):
        red_v[pl.ds(i * 16, 16)] = z16

    def red_t(t, carry):
        pltpu.sync_copy(denall.at[c, t, pl.ds(s * _NPT, _NPT)], tmp_v)
        for i in range(_NPT // 16):
            sl = pl.ds(i * 16, 16)
            red_v[sl] = red_v[sl] + tmp_v[sl]
        return carry

    lax.fori_loop(0, 16, red_t, 0)
    pltpu.sync_copy(red_v, denp2.at[c, pl.ds(s * _NPT, _NPT)])


def _edge2(srcs, dsts, stats2, m2f):
    mesh = plsc.VectorSubcoreMesh(core_axis_name="c", subcore_axis_name="s")
    f = pl.kernel(
        _edge2_body,
        out_type=[
            jax.ShapeDtypeStruct((_EPAD,), _f32),
            jax.ShapeDtypeStruct((2, _NPAD), _f32),
            jax.ShapeDtypeStruct((2, 16, _NPAD), _f32),
        ],
        mesh=mesh,
        compiler_params=pltpu.CompilerParams(needs_layout_passes=False, use_tc_tiling_on_sc=False),
        scratch_types=[
            pltpu.VMEM((_NPAD * 4,), _f32),
            pltpu.VMEM((16,), _f32),
            pltpu.VMEM((_EPT,), _i32),
            pltpu.VMEM((_EPT,), _i32),
            pltpu.VMEM((_NPAD,), _f32),
            pltpu.VMEM((_P2C,), _f32),
            pltpu.VMEM((_NPT,), _f32),
            pltpu.VMEM((_NPT,), _f32),
            pltpu.SemaphoreType.DMA,
        ],
    )
    return f(srcs, dsts, stats2, m2f)


def _alpha_body(ex2, dstsf, denp2,
                alpha, dentot,
                den_v, tmp_full, dstf_v, exc_v, alc_v, sem):
    c = lax.axis_index("c")
    s = lax.axis_index("s")
    wid = c * 16 + s
    pltpu.sync_copy(denp2.at[0], den_v)
    pltpu.sync_copy(denp2.at[1], tmp_full)

    def addr(i, carry):
        sl = pl.ds(i * 16, 16)
        den_v[sl] = den_v[sl] + tmp_full[sl]
        return carry

    lax.fori_loop(0, _NPAD // 16, addr, 0)

    @pl.when(wid == 0)
    def _():
        pltpu.sync_copy(den_v, dentot)

    pltpu.sync_copy(dstsf.at[pl.ds(wid * _EPT, _EPT)], dstf_v)

    def chunk(j, carry):
        base = j * _P2C
        pltpu.sync_copy(ex2.at[pl.ds(wid * _EPT + base, _P2C)], exc_v)

        def grp(g, carry2):
            sl = pl.ds(g * 16, 16)
            dv = dstf_v[pl.ds(base + g * 16, 16)]
            dd = plsc.load_gather(den_v, [dv])
            alc_v[sl] = exc_v[sl] / (dd + 1e-16)
            return carry2

        lax.fori_loop(0, _P2C // 16, grp, 0)
        pltpu.sync_copy(alc_v, alpha.at[pl.ds(wid * _EPT + base, _P2C)])
        return carry

    lax.fori_loop(0, _NCH2, chunk, 0)


def _alpha_k(ex2, dsts, denp2):
    mesh = plsc.VectorSubcoreMesh(core_axis_name="c", subcore_axis_name="s")
    f = pl.kernel(
        _alpha_body,
        out_type=[
            jax.ShapeDtypeStruct((_EPAD,), _f32),
            jax.ShapeDtypeStruct((_NPAD,), _f32),
        ],
        mesh=mesh,
        compiler_params=pltpu.CompilerParams(needs_layout_passes=False, use_tc_tiling_on_sc=False),
        scratch_types=[
            pltpu.VMEM((_NPAD,), _f32),
            pltpu.VMEM((_NPAD,), _f32),
            pltpu.VMEM((_EPT,), _i32),
            pltpu.VMEM((_P2C,), _f32),
            pltpu.VMEM((_P2C,), _f32),
            pltpu.SemaphoreType.DMA,
        ],
    )
    return f(ex2, dsts, denp2)


def kernel(tree_feature, edge_index, num_tree_nodes, ptr,
           ln1_g, ln1_b, W_tree, b_tree, ln2_g, ln2_b, W_var, b_var,
           W_glob, b_glob, glbNode, W_g1, att_src1, att_dst1, bias_g1,
           W_g2, att_src2, att_dst2, bias_g2):
    loop = jnp.arange(_N, dtype=_i32)
    parts = []
    for b in range(_BATCH):
        nodes = ptr[b] + jnp.arange(_PER, dtype=_i32)
        g = jnp.full((_PER,), _N_NODES + b, dtype=_i32)
        parts.append(jnp.stack([g, nodes]))
        parts.append(jnp.stack([nodes, g]))
    ei = jnp.concatenate(
        [edge_index.astype(_i32)] + parts + [jnp.stack([loop, loop])], axis=1)
    padv = jnp.full((_EPAD - _E,), _NST - 1, dtype=_i32)
    srcs = jnp.concatenate([ei[0], padv])
    dsts = jnp.concatenate([ei[1], padv])

    r1 = lambda a: a.reshape(1, -1)
    x0 = _pre(tree_feature, r1(ln1_g), r1(ln1_b), W_tree, r1(b_tree),
              r1(ln2_g), r1(ln2_b), W_var, r1(b_var), W_glob, r1(b_glob))
    x = jnp.concatenate(
        [x0, jnp.broadcast_to(glbNode[None], (_BATCH, _HID)),
         jnp.zeros((_NPAD - _N, _HID), _f32)], axis=0)

    z64 = jnp.zeros((_HID,), _f32)
    att1 = jnp.stack([
        jnp.concatenate([att_src1[0], z64]),
        jnp.concatenate([z64, att_src1[1]]),
        jnp.concatenate([att_dst1[0], z64]),
        jnp.concatenate([z64, att_dst1[1]]),
    ], axis=1)
    xp1, stats1, m1 = _proj(x, W_g1, att1)
    m1f = jnp.concatenate([jnp.broadcast_to(m1[0, 0], (16,)),
                           jnp.broadcast_to(m1[0, 1], (16,))])

    exw, denp1, _da = _ex1(srcs, dsts, stats1[:_NST].reshape(-1), m1f)
    accp = _agg(srcs, dsts, exw, xp1)

    att2 = jnp.stack([att_src2[0], att_dst2[0], z64, z64], axis=1)
    xp2, stats2, m2 = _mid(accp, denp1.reshape(2, _NPAD, 2),
                           r1(bias_g1), W_g2, att2)
    m2f = jnp.broadcast_to(m2[0, 0], (16,))

    ex2, denp2, _db = _edge2(srcs, dsts, stats2.reshape(-1), m2f)
    alpha_full, dentot = _alpha_k(ex2, dsts, denp2)
    alpha = alpha_full[:_E][:, None]

    ngl = 2 * _PER * _BATCH
    ex2g = ex2[_NE:_NE + ngl].reshape(_BATCH, 2, _PER)[:, 1, :]
    ex2g = jnp.pad(ex2g, ((0, 0), (0, 30)))
    ex2self = ex2[_NE + ngl + _N_NODES:_NE + ngl + _N]
    xg = jnp.pad(xp2[:_N_NODES].reshape(_BATCH, _PER, _HID),
                 ((0, 0), (0, 30), (0, 64)))
    xself = jnp.pad(xp2[_N_NODES:_N], ((0, 0), (0, 64)))
    exselfb = jnp.broadcast_to(ex2self[:, None], (_BATCH, 128))
    deng = denp2[0, _N_NODES:_N] + denp2[1, _N_NODES:_N]
    dgb = jnp.broadcast_to(deng[:, None], (_BATCH, 128))
    b2 = jnp.pad(bias_g2, (0, 64)).reshape(1, 128)
    tg = _glb(ex2g, xg, xself, exselfb, dgb, b2)
    tree_glb = tg[:, :_HID]
    return (tree_glb, (ei, alpha))


# speedup vs baseline: 98.4489x; 1.0338x over previous
import functools
import jax
import jax.numpy as jnp
from jax import lax
from jax.experimental import pallas as pl
from jax.experimental.pallas import tpu as pltpu
from jax.experimental.pallas import tpu_sc as plsc

_N_NODES = 10000
_BATCH = 8
_PER = 1250
_FEAT = 128
_VAR = 25
_HID = 64
_NE = 320000
_N = _N_NODES + _BATCH
_NPAD = 10240
_NPT = _NPAD // 16
_E = _NE + 2 * _PER * _BATCH + _N
_EPAD = 350208
_EPT = _EPAD // 32
_NCH1 = _EPT // 64
_P2C = 1824
_NCH2 = _EPT // _P2C

_HI = lax.Precision.HIGHEST
_f32 = jnp.float32
_i32 = jnp.int32


def _splat16(v, idx):
    dn = lax.GatherDimensionNumbers(offset_dims=(), collapsed_slice_dims=(0,),
                                    start_index_map=(0,))
    return lax.gather(v, idx[:, None], dimension_numbers=dn, slice_sizes=(1,),
                      mode=lax.GatherScatterMode.PROMISE_IN_BOUNDS)


def _dot(a, b):
    return jnp.dot(a, b, precision=_HI, preferred_element_type=_f32)


def _pre_body(tf_ref, l1g, l1b, wt, bt, l2g, l2b, wv, bv, wg, bg, out_ref):
    blk = tf_ref[...]
    t = blk[:, :_FEAT]
    v = blk[:, _FEAT:]
    mu = jnp.mean(t, axis=-1, keepdims=True)
    va = jnp.mean((t - mu) ** 2, axis=-1, keepdims=True)
    tn = (t - mu) * lax.rsqrt(va + 1e-5) * l1g[...] + l1b[...]
    mu2 = jnp.mean(v, axis=-1, keepdims=True)
    va2 = jnp.mean((v - mu2) ** 2, axis=-1, keepdims=True)
    vn = (v - mu2) * lax.rsqrt(va2 + 1e-5) * l2g[...] + l2b[...]
    a = _dot(vn, wv[...]) + bv[...]
    b = _dot(tn, wt[...]) + bt[...]
    out_ref[...] = _dot(jnp.concatenate([a, b], axis=1), wg[...]) + bg[...]


def _pre(tfeat, l1g, l1b, wt, bt, l2g, l2b, wv, bv, wg, bg):
    nb = 2000
    full = lambda shp: pl.BlockSpec(shp, lambda i: (0, 0))
    return pl.pallas_call(
        _pre_body,
        grid=(_N_NODES // nb,),
        in_specs=[
            pl.BlockSpec((nb, _FEAT + _VAR), lambda i: (i, 0)),
            full((1, _FEAT)), full((1, _FEAT)),
            full((_FEAT, _HID)), full((1, _HID)),
            full((1, _VAR)), full((1, _VAR)),
            full((_VAR, _HID)), full((1, _HID)),
            full((2 * _HID, _HID)), full((1, _HID)),
        ],
        out_specs=pl.BlockSpec((nb, _HID), lambda i: (i, 0)),
        out_shape=jax.ShapeDtypeStruct((_N_NODES, _HID), _f32),
    )(tfeat, l1g, l1b, wt, bt, l2g, l2b, wv, bv, wg, bg)


def _proj_body(nsteps, x_ref, w_ref, att_ref, xp_ref, st_ref, m_ref, msc):
    i = pl.program_id(0)
    xp = _dot(x_ref[...], w_ref[...])
    xp_ref[...] = xp
    st = _dot(xp, att_ref[...])
    st_ref[...] = st
    cur = jnp.broadcast_to(jnp.max(st, axis=0)[None, :], (8, 4))

    @pl.when(i == 0)
    def _():
        msc[...] = cur

    @pl.when(i > 0)
    def _():
        msc[...] = jnp.maximum(msc[...], cur)

    @pl.when(i == nsteps - 1)
    def _():
        m_ref[...] = msc[...]


def _proj(x, w, att):
    nb = 2560
    nsteps = _NPAD // nb
    din, dout = w.shape
    return pl.pallas_call(
        functools.partial(_proj_body, nsteps),
        grid=(nsteps,),
        in_specs=[
            pl.BlockSpec((nb, din), lambda i: (i, 0)),
            pl.BlockSpec((din, dout), lambda i: (0, 0)),
            pl.BlockSpec((dout, 4), lambda i: (0, 0)),
        ],
        out_specs=[
            pl.BlockSpec((nb, dout), lambda i: (i, 0)),
            pl.BlockSpec((nb, 4), lambda i: (i, 0)),
            pl.BlockSpec((8, 4), lambda i: (0, 0)),
        ],
        out_shape=[
            jax.ShapeDtypeStruct((_NPAD, dout), _f32),
            jax.ShapeDtypeStruct((_NPAD, 4), _f32),
            jax.ShapeDtypeStruct((8, 4), _f32),
        ],
        scratch_shapes=[pltpu.VMEM((8, 4), _f32)],
    )(x, w, att)


def _mid_body(nsteps, a_ref, d_ref, b1, w2, att2, xp_ref, st_ref, m_ref, msc):
    i = pl.program_id(0)
    a = a_ref[...]
    acc = a[0] + a[1]
    d = d_ref[...]
    den = d[0] + d[1]
    h = jnp.concatenate(
        [acc[:, :_HID] / (den[:, 0:1] + 1e-16),
         acc[:, _HID:] / (den[:, 1:2] + 1e-16)], axis=1)
    h = jnp.maximum(h + b1[...], 0.0)
    xp2 = _dot(h, w2[...])
    xp_ref[...] = xp2
    st = _dot(xp2, att2[...])
    st_ref[...] = st
    cur = jnp.broadcast_to(jnp.max(st, axis=0)[None, :], (8, 4))

    @pl.when(i == 0)
    def _():
        msc[...] = cur

    @pl.when(i > 0)
    def _():
        msc[...] = jnp.maximum(msc[...], cur)

    @pl.when(i == nsteps - 1)
    def _():
        m_ref[...] = msc[...]


def _mid(accp, denp, b1, w2, att2):
    nb = 2560
    nsteps = _NPAD // nb
    return pl.pallas_call(
        functools.partial(_mid_body, nsteps),
        grid=(nsteps,),
        in_specs=[
            pl.BlockSpec((2, nb, 2 * _HID), lambda i: (0, i, 0)),
            pl.BlockSpec((2, nb, 2), lambda i: (0, i, 0)),
            pl.BlockSpec((1, 2 * _HID), lambda i: (0, 0)),
            pl.BlockSpec((2 * _HID, _HID), lambda i: (0, 0)),
            pl.BlockSpec((_HID, 4), lambda i: (0, 0)),
        ],
        out_specs=[
            pl.BlockSpec((nb, _HID), lambda i: (i, 0)),
            pl.BlockSpec((nb, 4), lambda i: (i, 0)),
            pl.BlockSpec((8, 4), lambda i: (0, 0)),
        ],
        out_shape=[
            jax.ShapeDtypeStruct((_NPAD, _HID), _f32),
            jax.ShapeDtypeStruct((_NPAD, 4), _f32),
            jax.ShapeDtypeStruct((8, 4), _f32),
        ],
        scratch_shapes=[pltpu.VMEM((8, 4), _f32)],
    )(accp, denp, b1, w2, att2)


def _glb_body(ex_ref, xg_ref, xs_ref, es_ref, dg_ref, b2, out_ref):
    ex = ex_ref[...]
    xg = xg_ref[...]
    num = jnp.concatenate(
        [_dot(ex[b:b + 1, :], xg[b]) for b in range(_BATCH)], axis=0)
    num = num + es_ref[...] * xs_ref[...]
    out_ref[...] = num / (dg_ref[...] + 1e-16) + b2[...]


def _glb(ex2g, xg, xself, exself, dgb, b2):
    return pl.pallas_call(
        _glb_body,
        out_shape=jax.ShapeDtypeStruct((_BATCH, 128), _f32),
    )(ex2g, xg, xself, exself, dgb, b2)


_NST = 10016
_P1C = 5472
_AGC = 96
_NAG = _EPT // _AGC
_NC0 = 146
_NC1 = 2 * _NAG - _NC0


def _ex1_body(srcs, dstsf, stats1, m1f,
              exw, denp1, denall,
              stats_v, m_v, sidx1_v, didx1_v, exc0_v, exc1_v, den_v,
              red_v, tmp_v, sem):
    c = lax.axis_index("c")
    s = lax.axis_index("s")
    wid = c * 16 + s
    ebase = wid * _EPT
    pltpu.sync_copy(stats1, stats_v)
    pltpu.sync_copy(m1f, m_v)
    m0 = m_v[pl.ds(0, 16)]
    m1 = m_v[pl.ds(16, 16)]
    z16 = jnp.zeros((16,), _f32)

    def zden(i, carry):
        den_v[pl.ds(i * 16, 16)] = z16
        return carry
    lax.fori_loop(0, 2 * _NPAD // 16, zden, 0)

    def exchunk(j, carry):
        base = j * _P1C
        pltpu.sync_copy(srcs.at[pl.ds(ebase + base, _P1C)], sidx1_v)
        pltpu.sync_copy(dstsf.at[pl.ds(ebase + base, _P1C)], didx1_v)

        def grp(g, carry2):
            off = g * 16
            sv = sidx1_v[pl.ds(off, 16)]
            dv = didx1_v[pl.ds(off, 16)]
            sv4 = sv * 4
            dv4 = dv * 4
            a0 = plsc.load_gather(stats_v, [sv4])
            a1 = plsc.load_gather(stats_v, [sv4 + 1])
            d0 = plsc.load_gather(stats_v, [dv4 + 2])
            d1 = plsc.load_gather(stats_v, [dv4 + 3])
            l0 = a0 + d0
            l0 = jnp.where(l0 > 0, l0, l0 * 0.2)
            e0 = jnp.exp(l0 - jnp.maximum(m0 + d0, 0.0))
            l1 = a1 + d1
            l1 = jnp.where(l1 > 0, l1, l1 * 0.2)
            e1 = jnp.exp(l1 - jnp.maximum(m1 + d1, 0.0))
            exc0_v[pl.ds(off, 16)] = e0
            exc1_v[pl.ds(off, 16)] = e1
            di = dv * 2
            plsc.addupdate_scatter(den_v, [di], e0)
            plsc.addupdate_scatter(den_v, [di + 1], e1)
            return carry2

        lax.fori_loop(0, _P1C // 16, grp, 0)
        pltpu.sync_copy(exc0_v, exw.at[0, pl.ds(ebase + base, _P1C)])
        pltpu.sync_copy(exc1_v, exw.at[1, pl.ds(ebase + base, _P1C)])
        return carry

    lax.fori_loop(0, _EPT // _P1C, exchunk, 0)

    pltpu.sync_copy(den_v, denall.at[c, s])
    plsc.subcore_barrier()
    for half in range(2):
        hb = s * 1280 + half * 640
        for i in range(40):
            red_v[pl.ds(i * 16, 16)] = z16
        pltpu.sync_copy(denall.at[c, :, pl.ds(hb, 640)], tmp_v)
        for t in range(16):
            for i in range(40):
                sl = pl.ds(i * 16, 16)
                red_v[sl] = red_v[sl] + tmp_v[t, sl]
        pltpu.sync_copy(red_v, denp1.at[c, pl.ds(hb, 640)])


def _ex1(srcs, dsts, stats1, m1f):
    mesh = plsc.VectorSubcoreMesh(core_axis_name="c", subcore_axis_name="s")
    f = pl.kernel(
        _ex1_body,
        out_type=[
            jax.ShapeDtypeStruct((2, _EPAD), _f32),
            jax.ShapeDtypeStruct((2, 2 * _NPAD), _f32),
            jax.ShapeDtypeStruct((2, 16, 2 * _NPAD), _f32),
        ],
        mesh=mesh,
        compiler_params=pltpu.CompilerParams(
            needs_layout_passes=False, use_tc_tiling_on_sc=False),
        scratch_types=[
            pltpu.VMEM((_NST * 4,), _f32),
            pltpu.VMEM((32,), _f32),
            pltpu.VMEM((_P1C,), _i32),
            pltpu.VMEM((_P1C,), _i32),
            pltpu.VMEM((_P1C,), _f32),
            pltpu.VMEM((_P1C,), _f32),
            pltpu.VMEM((2 * _NPAD,), _f32),
            pltpu.VMEM((640,), _f32),
            pltpu.VMEM((16, 640), _f32),
            pltpu.SemaphoreType.DMA,
        ],
    )
    return f(srcs, dsts, stats1, m1f)


def _agg_body(srcs, dstsf, exw, xp1,
              accp,
              sA, dA, e0A, e1A, sB, dB, e0B, e1B, dsA, dsB,
              rowsA, rowsB,
              acc_sh,
              sem_iA, sem_iB, sem_gA, sem_gB, sem_sA, sem_sB):
    c = lax.axis_index("c")
    s = lax.axis_index("s")
    nch = jnp.where(c == 0, _NC0, _NC1)
    ebase = jnp.where(c == 0, s * (_NC0 * _AGC),
                      16 * (_NC0 * _AGC) + s * (_NC1 * _AGC))
    z16 = jnp.zeros((16,), _f32)

    for k in range(_AGC):
        for f in range(8):
            rowsA[k, pl.ds(f * 16, 16)] = z16
    nfull = _NPT // _AGC
    for k in range(nfull):
        pltpu.sync_copy(rowsA, acc_sh.at[pl.ds(s * _NPT + k * _AGC, _AGC)])
    rem = _NPT - nfull * _AGC
    if rem:
        pltpu.sync_copy(rowsA.at[pl.ds(0, rem)],
                        acc_sh.at[pl.ds(s * _NPT + nfull * _AGC, rem)])
    plsc.subcore_barrier()

    bufs = ((sA, dA, e0A, e1A, dsA, rowsA, sem_iA, sem_gA, sem_sA),
            (sB, dB, e0B, e1B, dsB, rowsB, sem_iB, sem_gB, sem_sB))
    kkc = [jnp.full((16,), t, _i32) for t in range(16)]

    def issue_idx(P, j):
        (si, di, e0, e1, _, _, sem_i, _, _) = bufs[P]
        off = ebase + j * _AGC
        pltpu.async_copy(srcs.at[pl.ds(off, _AGC)], si, sem_i)
        pltpu.async_copy(dstsf.at[pl.ds(off, _AGC)], di, sem_i)
        pltpu.async_copy(exw.at[0, pl.ds(off, _AGC)], e0, sem_i)
        pltpu.async_copy(exw.at[1, pl.ds(off, _AGC)], e1, sem_i)

    def wait_idx(P, j):
        (si, di, e0, e1, _, _, sem_i, _, _) = bufs[P]
        off = ebase + j * _AGC
        pltpu.make_async_copy(srcs.at[pl.ds(off, _AGC)], si, sem_i).wait()
        pltpu.make_async_copy(dstsf.at[pl.ds(off, _AGC)], di, sem_i).wait()
        pltpu.make_async_copy(exw.at[0, pl.ds(off, _AGC)], e0, sem_i).wait()
        pltpu.make_async_copy(exw.at[1, pl.ds(off, _AGC)], e1, sem_i).wait()

    def issue_gather(P):
        (si, _, _, _, _, rows, _, sem_g, _) = bufs[P]
        pltpu.async_copy(xp1.at[si], rows, sem_g)

    def wait_gather(P):
        (si, _, _, _, _, rows, _, sem_g, _) = bufs[P]
        pltpu.make_async_copy(xp1.at[si], rows, sem_g).wait()

    def issue_scatter(P):
        (_, _, _, _, dsc, rows, _, _, sem_s) = bufs[P]
        pltpu.async_copy(rows, acc_sh.at[dsc], sem_s, add=True)

    def wait_scatter(P):
        (_, _, _, _, dsc, rows, _, _, sem_s) = bufs[P]
        pltpu.make_async_copy(rows, acc_sh.at[dsc], sem_s).wait()

    def sub(P, j):
        (si, di, e0, e1, dsc, rows, _, _, _) = bufs[P]
        Q = 1 - P

        @pl.when(j >= 1)
        def _():
            wait_scatter(Q)

        @pl.when(j <= nch - 2)
        def _():
            wait_idx(Q, j + 1)
            issue_gather(Q)

        wait_gather(P)
        for g in range(_AGC // 16):
            ev0 = e0[pl.ds(g * 16, 16)]
            ev1 = e1[pl.ds(g * 16, 16)]
            for kk in range(16):
                k = g * 16 + kk
                eb0 = _splat16(ev0, kkc[kk])
                eb1 = _splat16(ev1, kkc[kk])
                for f in range(4):
                    sl = pl.ds(f * 16, 16)
                    rows[k, sl] = rows[k, sl] * eb0
                for f in range(4, 8):
                    sl = pl.ds(f * 16, 16)
                    rows[k, sl] = rows[k, sl] * eb1
        for g in range(_AGC // 16):
            sl = pl.ds(g * 16, 16)
            dsc[sl] = di[sl]
        issue_scatter(P)

        @pl.when(j <= nch - 3)
        def _():
            issue_idx(P, j + 2)

    issue_idx(0, 0)
    issue_idx(1, 1)
    wait_idx(0, 0)
    issue_gather(0)

    def pair(i, carry):
        sub(0, 2 * i)
        sub(1, 2 * i + 1)
        return carry

    lax.fori_loop(0, nch // 2, pair, 0)
    wait_scatter(1)
    plsc.subcore_barrier()
    pltpu.sync_copy(acc_sh.at[pl.ds(s * _NPT, _NPT)],
                    accp.at[c, pl.ds(s * _NPT, _NPT)])


def _agg(srcs, dsts, exw, xp1):
    mesh = plsc.VectorSubcoreMesh(core_axis_name="c", subcore_axis_name="s")
    f = pl.kernel(
        _agg_body,
        out_type=jax.ShapeDtypeStruct((2, _NPAD, 2 * _HID), _f32),
        mesh=mesh,
        compiler_params=pltpu.CompilerParams(
            needs_layout_passes=False, use_tc_tiling_on_sc=False),
        scratch_types=[
            pltpu.VMEM((_AGC,), _i32), pltpu.VMEM((_AGC,), _i32),
            pltpu.VMEM((_AGC,), _f32), pltpu.VMEM((_AGC,), _f32),
            pltpu.VMEM((_AGC,), _i32), pltpu.VMEM((_AGC,), _i32),
            pltpu.VMEM((_AGC,), _f32), pltpu.VMEM((_AGC,), _f32),
            pltpu.VMEM((_AGC,), _i32), pltpu.VMEM((_AGC,), _i32),
            pltpu.VMEM((_AGC, 2 * _HID), _f32),
            pltpu.VMEM((_AGC, 2 * _HID), _f32),
            pltpu.VMEM_SHARED((_NPAD, 2 * _HID), _f32),
            pltpu.SemaphoreType.DMA, pltpu.SemaphoreType.DMA,
            pltpu.SemaphoreType.DMA, pltpu.SemaphoreType.DMA,
            pltpu.SemaphoreType.DMA, pltpu.SemaphoreType.DMA,
        ],
    )
    return f(srcs, dsts, exw, xp1)


def _edge2_body(srcs, dstsf, stats2, m2f,
                ex2, denp2, denall,
                stats_v, m_v, src_v, dstf_v, den_v, exc_v, red_v, tmp_v,
                sem):
    c = lax.axis_index("c")
    s = lax.axis_index("s")
    wid = c * 16 + s
    pltpu.sync_copy(stats2, stats_v)
    pltpu.sync_copy(m2f, m_v)
    pltpu.sync_copy(srcs.at[pl.ds(wid * _EPT, _EPT)], src_v)
    pltpu.sync_copy(dstsf.at[pl.ds(wid * _EPT, _EPT)], dstf_v)
    m2 = m_v[pl.ds(0, 16)]
    z16 = jnp.zeros((16,), _f32)

    def zden(i, carry):
        den_v[pl.ds(i * 16, 16)] = z16
        return carry
    lax.fori_loop(0, _NPAD // 16, zden, 0)

    def chunk(j, carry):
        base = j * _P2C

        def grp(g, carry2):
            off = base + g * 16
            sv = src_v[pl.ds(off, 16)]
            dv = dstf_v[pl.ds(off, 16)]
            a = plsc.load_gather(stats_v, [sv * 4])
            d = plsc.load_gather(stats_v, [dv * 4 + 1])
            l = a + d
            l = jnp.where(l > 0, l, l * 0.2)
            e = jnp.exp(l - jnp.maximum(m2 + d, 0.0))
            exc_v[pl.ds(g * 16, 16)] = e
            plsc.addupdate_scatter(den_v, [dv], e)
            return carry2

        lax.fori_loop(0, _P2C // 16, grp, 0)
        pltpu.sync_copy(exc_v, ex2.at[pl.ds(wid * _EPT + base, _P2C)])
        return carry

    lax.fori_loop(0, _NCH2, chunk, 0)
    pltpu.sync_copy(den_v, denall.at[c, s])
    plsc.subcore_barrier()
    for i in range(_NPT // 16):
        red_v[pl.ds(i * 16, 16)] = z16
    pltpu.sync_copy(denall.at[c, :, pl.ds(s * _NPT, _NPT)], tmp_v)
    for t in range(16):
        for i in range(_NPT // 16):
            sl = pl.ds(i * 16, 16)
            red_v[sl] = red_v[sl] + tmp_v[t, sl]
    pltpu.sync_copy(red_v, denp2.at[c, pl.ds(s * _NPT, _NPT)])


def _edge2(srcs, dsts, stats2, m2f):
    mesh = plsc.VectorSubcoreMesh(core_axis_name="c", subcore_axis_name="s")
    f = pl.kernel(
        _edge2_body,
        out_type=[
            jax.ShapeDtypeStruct((_EPAD,), _f32),
            jax.ShapeDtypeStruct((2, _NPAD), _f32),
            jax.ShapeDtypeStruct((2, 16, _NPAD), _f32),
        ],
        mesh=mesh,
        compiler_params=pltpu.CompilerParams(needs_layout_passes=False, use_tc_tiling_on_sc=False),
        scratch_types=[
            pltpu.VMEM((_NPAD * 4,), _f32),
            pltpu.VMEM((16,), _f32),
            pltpu.VMEM((_EPT,), _i32),
            pltpu.VMEM((_EPT,), _i32),
            pltpu.VMEM((_NPAD,), _f32),
            pltpu.VMEM((_P2C,), _f32),
            pltpu.VMEM((_NPT,), _f32),
            pltpu.VMEM((16, _NPT), _f32),
            pltpu.SemaphoreType.DMA,
        ],
    )
    return f(srcs, dsts, stats2, m2f)


def _alpha_body(ex2, dstsf, denp2,
                alpha, dentot,
                den_v, tmp_full, dstf_v, exc_v, alc_v, sem):
    c = lax.axis_index("c")
    s = lax.axis_index("s")
    wid = c * 16 + s
    pltpu.sync_copy(denp2.at[0], den_v)
    pltpu.sync_copy(denp2.at[1], tmp_full)

    def addr(i, carry):
        sl = pl.ds(i * 16, 16)
        den_v[sl] = den_v[sl] + tmp_full[sl]
        return carry

    lax.fori_loop(0, _NPAD // 16, addr, 0)

    @pl.when(wid == 0)
    def _():
        pltpu.sync_copy(den_v, dentot)

    pltpu.sync_copy(dstsf.at[pl.ds(wid * _EPT, _EPT)], dstf_v)

    def chunk(j, carry):
        base = j * _P2C
        pltpu.sync_copy(ex2.at[pl.ds(wid * _EPT + base, _P2C)], exc_v)

        def grp(g, carry2):
            sl = pl.ds(g * 16, 16)
            dv = dstf_v[pl.ds(base + g * 16, 16)]
            dd = plsc.load_gather(den_v, [dv])
            alc_v[sl] = exc_v[sl] / (dd + 1e-16)
            return carry2

        lax.fori_loop(0, _P2C // 16, grp, 0)
        pltpu.sync_copy(alc_v, alpha.at[pl.ds(wid * _EPT + base, _P2C)])
        return carry

    lax.fori_loop(0, _NCH2, chunk, 0)


def _alpha_k(ex2, dsts, denp2):
    mesh = plsc.VectorSubcoreMesh(core_axis_name="c", subcore_axis_name="s")
    f = pl.kernel(
        _alpha_body,
        out_type=[
            jax.ShapeDtypeStruct((_EPAD,), _f32),
            jax.ShapeDtypeStruct((_NPAD,), _f32),
        ],
        mesh=mesh,
        compiler_params=pltpu.CompilerParams(needs_layout_passes=False, use_tc_tiling_on_sc=False),
        scratch_types=[
            pltpu.VMEM((_NPAD,), _f32),
            pltpu.VMEM((_NPAD,), _f32),
            pltpu.VMEM((_EPT,), _i32),
            pltpu.VMEM((_P2C,), _f32),
            pltpu.VMEM((_P2C,), _f32),
            pltpu.SemaphoreType.DMA,
        ],
    )
    return f(ex2, dsts, denp2)


def kernel(tree_feature, edge_index, num_tree_nodes, ptr,
           ln1_g, ln1_b, W_tree, b_tree, ln2_g, ln2_b, W_var, b_var,
           W_glob, b_glob, glbNode, W_g1, att_src1, att_dst1, bias_g1,
           W_g2, att_src2, att_dst2, bias_g2):
    loop = jnp.arange(_N, dtype=_i32)
    parts = []
    for b in range(_BATCH):
        nodes = ptr[b] + jnp.arange(_PER, dtype=_i32)
        g = jnp.full((_PER,), _N_NODES + b, dtype=_i32)
        parts.append(jnp.stack([g, nodes]))
        parts.append(jnp.stack([nodes, g]))
    ei = jnp.concatenate(
        [edge_index.astype(_i32)] + parts + [jnp.stack([loop, loop])], axis=1)
    padv = jnp.full((_EPAD - _E,), _NST - 1, dtype=_i32)
    srcs = jnp.concatenate([ei[0], padv])
    dsts = jnp.concatenate([ei[1], padv])

    r1 = lambda a: a.reshape(1, -1)
    x0 = _pre(tree_feature, r1(ln1_g), r1(ln1_b), W_tree, r1(b_tree),
              r1(ln2_g), r1(ln2_b), W_var, r1(b_var), W_glob, r1(b_glob))
    x = jnp.concatenate(
        [x0, jnp.broadcast_to(glbNode[None], (_BATCH, _HID)),
         jnp.zeros((_NPAD - _N, _HID), _f32)], axis=0)

    z64 = jnp.zeros((_HID,), _f32)
    att1 = jnp.stack([
        jnp.concatenate([att_src1[0], z64]),
        jnp.concatenate([z64, att_src1[1]]),
        jnp.concatenate([att_dst1[0], z64]),
        jnp.concatenate([z64, att_dst1[1]]),
    ], axis=1)
    xp1, stats1, m1 = _proj(x, W_g1, att1)
    m1f = jnp.concatenate([jnp.broadcast_to(m1[0, 0], (16,)),
                           jnp.broadcast_to(m1[0, 1], (16,))])

    exw, denp1, _da = _ex1(srcs, dsts, stats1[:_NST].reshape(-1), m1f)
    accp = _agg(srcs, dsts, exw, xp1)

    att2 = jnp.stack([att_src2[0], att_dst2[0], z64, z64], axis=1)
    xp2, stats2, m2 = _mid(accp, denp1.reshape(2, _NPAD, 2),
                           r1(bias_g1), W_g2, att2)
    m2f = jnp.broadcast_to(m2[0, 0], (16,))

    ex2, denp2, _db = _edge2(srcs, dsts, stats2.reshape(-1), m2f)
    alpha_full, dentot = _alpha_k(ex2, dsts, denp2)
    alpha = alpha_full[:_E][:, None]

    ngl = 2 * _PER * _BATCH
    ex2g = ex2[_NE:_NE + ngl].reshape(_BATCH, 2, _PER)[:, 1, :]
    ex2g = jnp.pad(ex2g, ((0, 0), (0, 30)))
    ex2self = ex2[_NE + ngl + _N_NODES:_NE + ngl + _N]
    xg = jnp.pad(xp2[:_N_NODES].reshape(_BATCH, _PER, _HID),
                 ((0, 0), (0, 30), (0, 64)))
    xself = jnp.pad(xp2[_N_NODES:_N], ((0, 0), (0, 64)))
    exselfb = jnp.broadcast_to(ex2self[:, None], (_BATCH, 128))
    deng = denp2[0, _N_NODES:_N] + denp2[1, _N_NODES:_N]
    dgb = jnp.broadcast_to(deng[:, None], (_BATCH, 128))
    b2 = jnp.pad(bias_g2, (0, 64)).reshape(1, 128)
    tg = _glb(ex2g, xg, xself, exselfb, dgb, b2)
    tree_glb = tg[:, :_HID]
    return (tree_glb, (ei, alpha))
